# baseline (speedup 1.0000x reference)
import jax
import jax.numpy as jnp
from jax import lax
from jax.experimental import pallas as pl
from jax.experimental.pallas import tpu as pltpu
from jax.experimental.pallas import tpu_sc as plsc

N_NODES = 10000
D = 128
E = 320000

NC = 2
NS = 16
NW = NC * NS
L = 16

EPW = E // NW
CH = 80
N_CHUNKS = EPW // CH
G = CH // L
FC = D // L


def _body(table, src_idx, dst_idx, out,
          idx_s, idx_d, rows_s0, rows_d0, rows_s1, rows_d1,
          out_v, stage, sem0, sem1):
    wid = lax.axis_index("s") * NC + lax.axis_index("c")
    base = wid * EPW

    pltpu.sync_copy(src_idx.at[pl.ds(base, EPW)], idx_s)
    pltpu.sync_copy(dst_idx.at[pl.ds(base, EPW)], idx_d)

    rows = ((rows_s0, rows_d0, sem0), (rows_s1, rows_d1, sem1))
    lane = lax.iota(jnp.int32, L)

    def launch(b, c):
        pass

    def wait(b):
        pass

    def compute(b, c):
        rs, rd, _ = rows[b]
        for g in range(G):
            for j in range(L):
                e = g * L + j
                p = rs[e, pl.ds(0, L)] * rd[e, pl.ds(0, L)]
                for fc in range(1, FC):
                    p = p + rs[e, pl.ds(fc * L, L)] * rd[e, pl.ds(fc * L, L)]
                plsc.store_scatter(stage, [lane * L + j], p)
            acc = stage[pl.ds(0, L)]
            for r in range(1, L):
                acc = acc + stage[pl.ds(r * L, L)]
            out_v[pl.ds(c * CH + g * L, L)] = acc

    launch(0, 0)

    def pair_body(i, _):
        t0 = 2 * i
        launch(1, t0 + 1)
        wait(0)
        compute(0, t0)
        launch(0, t0 + 2)
        wait(1)
        compute(1, t0 + 1)
        return 0

    lax.fori_loop(0, (N_CHUNKS - 1) // 2, pair_body, 0)
    wait(0)
    compute(0, N_CHUNKS - 1)

    pltpu.sync_copy(out_v, out.at[pl.ds(base, EPW)])


@jax.jit
def kernel(x_module, edge_label_index):
    src = edge_label_index[0]
    dst = edge_label_index[1]
    mesh = plsc.VectorSubcoreMesh(core_axis_name="c", subcore_axis_name="s")
    return pl.kernel(
        _body,
        out_type=jax.ShapeDtypeStruct((E,), jnp.float32),
        mesh=mesh,
        compiler_params=pltpu.CompilerParams(needs_layout_passes=False),
        scratch_types=[
            pltpu.VMEM((EPW,), jnp.int32),
            pltpu.VMEM((EPW,), jnp.int32),
            pltpu.VMEM((CH, D), jnp.float32),
            pltpu.VMEM((CH, D), jnp.float32),
            pltpu.VMEM((CH, D), jnp.float32),
            pltpu.VMEM((CH, D), jnp.float32),
            pltpu.VMEM((EPW,), jnp.float32),
            pltpu.VMEM((L * L,), jnp.float32),
            pltpu.SemaphoreType.DMA,
            pltpu.SemaphoreType.DMA,
        ],
    )(x_module, src, dst)


# speedup vs baseline: 1.0055x; 1.0055x over previous
import jax
import jax.numpy as jnp
from jax import lax
from jax.experimental import pallas as pl
from jax.experimental.pallas import tpu as pltpu
from jax.experimental.pallas import tpu_sc as plsc

N_NODES = 10000
D = 128
E = 320000

NC = 2
NS = 16
NW = NC * NS
L = 16

W = D // 2
EPW = E // NW
CH = 80
N_CHUNKS = EPW // CH
G = CH // L
FC = W // L
PS = L + 1


def _body(table, src_idx, dst_idx, out,
          idx_s, idx_d, rows_s0, rows_d0, rows_s1, rows_d1,
          out_v, stage, sem0, sem1):
    wid = lax.axis_index("s") * NC + lax.axis_index("c")
    base = wid * EPW

    pltpu.sync_copy(src_idx.at[pl.ds(base, EPW)], idx_s)
    pltpu.sync_copy(dst_idx.at[pl.ds(base, EPW)], idx_d)

    rows = ((rows_s0, rows_d0, sem0), (rows_s1, rows_d1, sem1))
    lane = lax.iota(jnp.int32, L)

    def launch(b, c):
        rs, rd, sem = rows[b]
        pltpu.async_copy(table.at[idx_s.at[pl.ds(c * CH, CH)]], rs, sem)
        pltpu.async_copy(table.at[idx_d.at[pl.ds(c * CH, CH)]], rd, sem)

    def wait(b):
        rs, rd, sem = rows[b]
        pltpu.make_async_copy(table.at[idx_s.at[pl.ds(0, CH)]], rs, sem).wait()
        pltpu.make_async_copy(table.at[idx_d.at[pl.ds(0, CH)]], rd, sem).wait()

    def compute(b, c):
        rs, rd, _ = rows[b]
        for g in range(G):
            for j in range(L):
                e = g * L + j
                acc = None
                for fc in range(FC):
                    sb = plsc.bitcast(rs[e, pl.ds(fc * L, L)], jnp.bfloat16)
                    db = plsc.bitcast(rd[e, pl.ds(fc * L, L)], jnp.bfloat16)
                    sa, sc = plsc.unpack(
                        sb, format=plsc.PackFormat.INTERLEAVED,
                        preferred_element_type=jnp.float32)
                    da, dc = plsc.unpack(
                        db, format=plsc.PackFormat.INTERLEAVED,
                        preferred_element_type=jnp.float32)
                    part = sa * da + sc * dc
                    acc = part if acc is None else acc + part
                plsc.store_scatter(stage, [lane * PS + j], acc)
            acc = stage[pl.ds(0, L)]
            for r in range(1, L):
                acc = acc + stage[pl.ds(r * PS, L)]
            out_v[pl.ds(c * CH + g * L, L)] = acc

    launch(0, 0)

    def pair_body(i, _):
        t0 = 2 * i
        launch(1, t0 + 1)
        wait(0)
        compute(0, t0)
        launch(0, t0 + 2)
        wait(1)
        compute(1, t0 + 1)
        return 0

    lax.fori_loop(0, (N_CHUNKS - 1) // 2, pair_body, 0)
    wait(0)
    compute(0, N_CHUNKS - 1)

    pltpu.sync_copy(out_v, out.at[pl.ds(base, EPW)])


@jax.jit
def kernel(x_module, edge_label_index):
    src = edge_label_index[0]
    dst = edge_label_index[1]
    packed = lax.bitcast_convert_type(
        x_module.astype(jnp.bfloat16).reshape(N_NODES, W, 2), jnp.int32
    )
    packed = jnp.concatenate(
        [packed, jnp.zeros((N_NODES, D - W), jnp.int32)], axis=1
    )
    mesh = plsc.VectorSubcoreMesh(core_axis_name="c", subcore_axis_name="s")
    return pl.kernel(
        _body,
        out_type=jax.ShapeDtypeStruct((E,), jnp.float32),
        mesh=mesh,
        compiler_params=pltpu.CompilerParams(needs_layout_passes=False),
        scratch_types=[
            pltpu.VMEM((EPW,), jnp.int32),
            pltpu.VMEM((EPW,), jnp.int32),
            pltpu.VMEM((CH, D), jnp.int32),
            pltpu.VMEM((CH, D), jnp.int32),
            pltpu.VMEM((CH, D), jnp.int32),
            pltpu.VMEM((CH, D), jnp.int32),
            pltpu.VMEM((EPW,), jnp.float32),
            pltpu.VMEM((L * PS,), jnp.float32),
            pltpu.SemaphoreType.DMA,
            pltpu.SemaphoreType.DMA,
        ],
    )(packed, src, dst)


# speedup vs baseline: 1.4366x; 1.4288x over previous
import jax
import jax.numpy as jnp
from jax import lax
from jax.experimental import pallas as pl
from jax.experimental.pallas import tpu as pltpu
from jax.experimental.pallas import tpu_sc as plsc

N_NODES = 10000
D = 128
E = 320000

NC = 2
NS = 16
NW = NC * NS
L = 16

W = D // 2
EPW = E // NW
CH = 80
N_CHUNKS = EPW // CH
G = CH // L
FC = W // L
PS = L + 1


def _body(table, src_idx, dst_idx, out,
          idx_s, idx_d, rows_s0, rows_d0, rows_s1, rows_d1,
          out_v, stage, sem0, sem1):
    wid = lax.axis_index("s") * NC + lax.axis_index("c")
    base = wid * EPW

    pltpu.sync_copy(src_idx.at[pl.ds(base, EPW)], idx_s)
    pltpu.sync_copy(dst_idx.at[pl.ds(base, EPW)], idx_d)

    rows = ((rows_s0, rows_d0, sem0), (rows_s1, rows_d1, sem1))
    lane = lax.iota(jnp.int32, L)

    def launch(b, c):
        rs, rd, sem = rows[b]
        pltpu.async_copy(table.at[idx_s.at[pl.ds(c * CH, CH)]], rs, sem)
        pltpu.async_copy(table.at[idx_d.at[pl.ds(c * CH, CH)]], rd, sem)

    def wait(b):
        rs, rd, sem = rows[b]
        pltpu.make_async_copy(table.at[idx_s.at[pl.ds(0, CH)]], rs, sem).wait()
        pltpu.make_async_copy(table.at[idx_d.at[pl.ds(0, CH)]], rd, sem).wait()

    def compute(b, c):
        rs, rd, _ = rows[b]
        for g in range(G):
            accs = []
            for j in range(L):
                e = g * L + j
                parts = []
                for fc in range(FC):
                    sb = plsc.bitcast(rs[e, pl.ds(fc * L, L)], jnp.bfloat16)
                    db = plsc.bitcast(rd[e, pl.ds(fc * L, L)], jnp.bfloat16)
                    sa, sc = plsc.unpack(
                        sb, format=plsc.PackFormat.INTERLEAVED,
                        preferred_element_type=jnp.float32)
                    da, dc = plsc.unpack(
                        db, format=plsc.PackFormat.INTERLEAVED,
                        preferred_element_type=jnp.float32)
                    parts.append(sa * da + sc * dc)
                accs.append((parts[0] + parts[1]) + (parts[2] + parts[3]))
            for j in range(L):
                plsc.store_scatter(stage, [lane * PS + j], accs[j])
            acc = stage[pl.ds(0, L)]
            for r in range(1, L):
                acc = acc + stage[pl.ds(r * PS, L)]
            out_v[pl.ds(c * CH + g * L, L)] = acc

    launch(0, 0)

    def pair_body(i, _):
        t0 = 2 * i
        launch(1, t0 + 1)
        wait(0)
        compute(0, t0)
        launch(0, t0 + 2)
        wait(1)
        compute(1, t0 + 1)
        return 0

    lax.fori_loop(0, (N_CHUNKS - 1) // 2, pair_body, 0)
    wait(0)
    compute(0, N_CHUNKS - 1)

    pltpu.sync_copy(out_v, out.at[pl.ds(base, EPW)])


@jax.jit
def kernel(x_module, edge_label_index):
    src = edge_label_index[0]
    dst = edge_label_index[1]
    packed = lax.bitcast_convert_type(
        x_module.astype(jnp.bfloat16).reshape(N_NODES, W, 2), jnp.int32
    )
    packed = jnp.concatenate(
        [packed, jnp.zeros((N_NODES, D - W), jnp.int32)], axis=1
    )
    mesh = plsc.VectorSubcoreMesh(core_axis_name="c", subcore_axis_name="s")
    return pl.kernel(
        _body,
        out_type=jax.ShapeDtypeStruct((E,), jnp.float32),
        mesh=mesh,
        compiler_params=pltpu.CompilerParams(needs_layout_passes=False),
        scratch_types=[
            pltpu.VMEM((EPW,), jnp.int32),
            pltpu.VMEM((EPW,), jnp.int32),
            pltpu.VMEM((CH, D), jnp.int32),
            pltpu.VMEM((CH, D), jnp.int32),
            pltpu.VMEM((CH, D), jnp.int32),
            pltpu.VMEM((CH, D), jnp.int32),
            pltpu.VMEM((EPW,), jnp.float32),
            pltpu.VMEM((L * PS,), jnp.float32),
            pltpu.SemaphoreType.DMA,
            pltpu.SemaphoreType.DMA,
        ],
    )(packed, src, dst)


# speedup vs baseline: 1.5591x; 1.0853x over previous
import jax
import jax.numpy as jnp
from jax import lax
from jax.experimental import pallas as pl
from jax.experimental.pallas import tpu as pltpu
from jax.experimental.pallas import tpu_sc as plsc

N_NODES = 10000
D = 128
E = 320000

NC = 2
NS = 16
NW = NC * NS
L = 16

W = D // 2
EPW = E // NW
CH = 80
N_CHUNKS = EPW // CH
G = CH // L
FC = W // L
PS = L + 1
NP = 10240


def _body(table, src_idx, dst_idx, out,
          idx_s, idx_d, rows_s0, rows_d0, rows_s1, rows_d1,
          out_v, stage, sem0, sem1):
    wid = lax.axis_index("s") * NC + lax.axis_index("c")
    base = wid * EPW

    pltpu.sync_copy(src_idx.at[pl.ds(base, EPW)], idx_s)
    pltpu.sync_copy(dst_idx.at[pl.ds(base, EPW)], idx_d)

    rows = ((rows_s0, rows_d0, sem0), (rows_s1, rows_d1, sem1))
    lane = lax.iota(jnp.int32, L)

    def launch(b, c):
        rs, rd, sem = rows[b]
        pltpu.async_copy(table.at[idx_s.at[pl.ds(c * CH, CH)]], rs, sem)
        pltpu.async_copy(table.at[idx_d.at[pl.ds(c * CH, CH)]], rd, sem)

    def wait(b):
        rs, rd, sem = rows[b]
        pltpu.make_async_copy(table.at[idx_s.at[pl.ds(0, CH)]], rs, sem).wait()
        pltpu.make_async_copy(table.at[idx_d.at[pl.ds(0, CH)]], rd, sem).wait()

    def compute(b, c):
        rs, rd, _ = rows[b]
        for g in range(G):
            accs = []
            for j in range(L):
                e = g * L + j
                parts = []
                for fc in range(FC):
                    sb = plsc.bitcast(rs[e, pl.ds(fc * L, L)], jnp.bfloat16)
                    db = plsc.bitcast(rd[e, pl.ds(fc * L, L)], jnp.bfloat16)
                    sa, sc = plsc.unpack(
                        sb, format=plsc.PackFormat.INTERLEAVED,
                        preferred_element_type=jnp.float32)
                    da, dc = plsc.unpack(
                        db, format=plsc.PackFormat.INTERLEAVED,
                        preferred_element_type=jnp.float32)
                    parts.append(sa * da + sc * dc)
                accs.append((parts[0] + parts[1]) + (parts[2] + parts[3]))
            for j in range(L):
                plsc.store_scatter(stage, [lane * PS + j], accs[j])
            acc = stage[pl.ds(0, L)]
            for r in range(1, L):
                acc = acc + stage[pl.ds(r * PS, L)]
            out_v[pl.ds(c * CH + g * L, L)] = acc

    launch(0, 0)

    def pair_body(i, _):
        t0 = 2 * i
        launch(1, t0 + 1)
        wait(0)
        compute(0, t0)
        launch(0, t0 + 2)
        wait(1)
        compute(1, t0 + 1)
        return 0

    lax.fori_loop(0, (N_CHUNKS - 1) // 2, pair_body, 0)
    wait(0)
    compute(0, N_CHUNKS - 1)

    pltpu.sync_copy(out_v, out.at[pl.ds(base, EPW)])


@jax.jit
def kernel(x_module, edge_label_index):
    src = edge_label_index[0]
    dst = edge_label_index[1]
    packed = lax.bitcast_convert_type(
        x_module.astype(jnp.bfloat16).reshape(N_NODES, W, 2), jnp.int32
    )
    packed = jnp.concatenate(
        [packed, jnp.zeros((NP - N_NODES, W), jnp.int32)], axis=0
    )
    mesh = plsc.VectorSubcoreMesh(core_axis_name="c", subcore_axis_name="s")
    return pl.kernel(
        _body,
        out_type=jax.ShapeDtypeStruct((E,), jnp.float32),
        mesh=mesh,
        compiler_params=pltpu.CompilerParams(needs_layout_passes=False, use_tc_tiling_on_sc=False),
        scratch_types=[
            pltpu.VMEM((EPW,), jnp.int32),
            pltpu.VMEM((EPW,), jnp.int32),
            pltpu.VMEM((CH, W), jnp.int32),
            pltpu.VMEM((CH, W), jnp.int32),
            pltpu.VMEM((CH, W), jnp.int32),
            pltpu.VMEM((CH, W), jnp.int32),
            pltpu.VMEM((EPW,), jnp.float32),
            pltpu.VMEM((L * PS,), jnp.float32),
            pltpu.SemaphoreType.DMA,
            pltpu.SemaphoreType.DMA,
        ],
    )(packed, src, dst)


# speedup vs baseline: 1.7919x; 1.1493x over previous
import jax
import jax.numpy as jnp
from jax import lax
from jax.experimental import pallas as pl
from jax.experimental.pallas import tpu as pltpu
from jax.experimental.pallas import tpu_sc as plsc

N_NODES = 10000
D = 128
E = 320000

NC = 2
NS = 16
NW = NC * NS
L = 16

W = D // 2
EPW = E // NW
CH = 80
N_CHUNKS = EPW // CH
G = CH // L
FC = W // L
PS = L + 1
NP = 10240


def _body(table, src_idx, dst_idx, out,
          idx_s, idx_d, rows_s0, rows_d0, rows_s1, rows_d1,
          out_v, stage, sem0, sem1):
    wid = lax.axis_index("s") * NC + lax.axis_index("c")
    base = wid * EPW

    pltpu.sync_copy(src_idx.at[pl.ds(base, EPW)], idx_s)
    pltpu.sync_copy(dst_idx.at[pl.ds(base, EPW)], idx_d)

    rows = ((rows_s0, rows_d0, sem0), (rows_s1, rows_d1, sem1))
    lane = lax.iota(jnp.int32, L)

    def launch(b, c):
        rs, rd, sem = rows[b]
        pltpu.async_copy(table.at[idx_s.at[pl.ds(c * CH, CH)]], rs, sem)
        pltpu.async_copy(table.at[idx_d.at[pl.ds(c * CH, CH)]], rd, sem)

    def wait(b):
        rs, rd, sem = rows[b]
        pltpu.make_async_copy(table.at[idx_s.at[pl.ds(0, CH)]], rs, sem).wait()
        pltpu.make_async_copy(table.at[idx_d.at[pl.ds(0, CH)]], rd, sem).wait()

    def compute(b, c):
        rs, rd, _ = rows[b]
        for g in range(G):
            accs = []
            for j in range(L):
                e = g * L + j
                parts = []
                for fc in range(FC):
                    sb = plsc.bitcast(rs[e, pl.ds(fc * L, L)], jnp.bfloat16)
                    db = plsc.bitcast(rd[e, pl.ds(fc * L, L)], jnp.bfloat16)
                    pa, pb = plsc.unpack(
                        sb * db, format=plsc.PackFormat.INTERLEAVED,
                        preferred_element_type=jnp.float32)
                    parts.append(pa + pb)
                accs.append((parts[0] + parts[1]) + (parts[2] + parts[3]))
            for j in range(L):
                plsc.store_scatter(stage, [lane * PS + j], accs[j])
            acc = stage[pl.ds(0, L)]
            for r in range(1, L):
                acc = acc + stage[pl.ds(r * PS, L)]
            out_v[pl.ds(c * CH + g * L, L)] = acc

    launch(0, 0)

    def pair_body(i, _):
        t0 = 2 * i
        launch(1, t0 + 1)
        wait(0)
        compute(0, t0)
        launch(0, t0 + 2)
        wait(1)
        compute(1, t0 + 1)
        return 0

    lax.fori_loop(0, (N_CHUNKS - 1) // 2, pair_body, 0)
    wait(0)
    compute(0, N_CHUNKS - 1)

    pltpu.sync_copy(out_v, out.at[pl.ds(base, EPW)])


@jax.jit
def kernel(x_module, edge_label_index):
    src = edge_label_index[0]
    dst = edge_label_index[1]
    packed = lax.bitcast_convert_type(
        x_module.astype(jnp.bfloat16).reshape(N_NODES, W, 2), jnp.int32
    )
    packed = jnp.concatenate(
        [packed, jnp.zeros((NP - N_NODES, W), jnp.int32)], axis=0
    )
    mesh = plsc.VectorSubcoreMesh(core_axis_name="c", subcore_axis_name="s")
    return pl.kernel(
        _body,
        out_type=jax.ShapeDtypeStruct((E,), jnp.float32),
        mesh=mesh,
        compiler_params=pltpu.CompilerParams(needs_layout_passes=False, use_tc_tiling_on_sc=False),
        scratch_types=[
            pltpu.VMEM((EPW,), jnp.int32),
            pltpu.VMEM((EPW,), jnp.int32),
            pltpu.VMEM((CH, W), jnp.int32),
            pltpu.VMEM((CH, W), jnp.int32),
            pltpu.VMEM((CH, W), jnp.int32),
            pltpu.VMEM((CH, W), jnp.int32),
            pltpu.VMEM((EPW,), jnp.float32),
            pltpu.VMEM((L * PS,), jnp.float32),
            pltpu.SemaphoreType.DMA,
            pltpu.SemaphoreType.DMA,
        ],
    )(packed, src, dst)
